# Initial kernel scaffold; baseline (speedup 1.0000x reference)
import functools

import jax
import jax.numpy as jnp
from jax import lax
from jax.experimental import pallas as pl
from jax.experimental.pallas import tpu as pltpu
from jax.experimental.pallas import tpu_sc as plsc

N = 100000
E = 1600000
D_IN = 16
D_HID = 64
D_OUT = 8

NP = 102400
ROW = 128
EROWS = 12544
EP = EROWS * ROW
PADNODE = NP - 1

NC = 2
NS = 16
L = 16

RPT64 = EROWS // NS
ST64 = 7
CH64 = RPT64 // ST64
RPT8 = EROWS // (NC * NS)
ST8 = 7
CH8 = RPT8 // ST8
NPT = NP // NS
DUMP = NPT // ROW

BLK = 2048
GRID = NP // BLK


def _deg_body(dstR, deg_out, degT, stage):
    c = lax.axis_index("c")
    s = lax.axis_index("s")
    w = c * NS + s
    zeros16 = jnp.zeros((L,), jnp.float32)
    ones16 = jnp.ones((L,), jnp.float32)

    @pl.loop(0, NP // L)
    def _(j):
        degT[pl.ds(j * L, L)] = zeros16

    rowbase = w * RPT8
    for st in range(ST8):
        pltpu.sync_copy(dstR.at[pl.ds(rowbase + st * CH8, CH8)], stage)

        @pl.loop(0, CH8)
        def _(j):
            row = stage.at[j]
            for k in range(ROW // L):
                idx = row[pl.ds(k * L, L)]
                plsc.addupdate_scatter(degT, [idx], ones16)

    pltpu.sync_copy(degT, deg_out.at[pl.ds(w * NP, NP)])


@functools.partial(
    pl.kernel,
    out_type=jax.ShapeDtypeStruct((NC * NS * NP,), jnp.float32),
    mesh=plsc.VectorSubcoreMesh(core_axis_name="c", subcore_axis_name="s"),
    scratch_types=[
        pltpu.VMEM((NP,), jnp.float32),
        pltpu.VMEM((CH8, ROW), jnp.int32),
    ],
)
def _sc_deg(dstR, deg_out, degT, stage):
    _deg_body(dstR, deg_out, degT, stage)


def _edge64_body(y, srcR, dstR, acc_out, accS, src_i, dst_i, rows, zbuf):
    c = lax.axis_index("c")
    s = lax.axis_index("s")
    zeros16 = jnp.zeros((L,), jnp.float32)

    @pl.loop(0, ROW)
    def _(j):
        zbuf[j] = zeros16

    rowbase = s * RPT64
    myacc = s * NPT
    for p in range(2):
        chunk = 2 * c + p
        off = chunk * NP

        @pl.loop(0, DUMP)
        def _(k):
            pltpu.sync_copy(zbuf, accS.at[pl.ds(myacc + k * ROW, ROW)])

        plsc.subcore_barrier()

        for st in range(ST64):
            r0 = rowbase + st * CH64
            pltpu.sync_copy(srcR.at[pl.ds(r0, CH64)], src_i)
            pltpu.sync_copy(dstR.at[pl.ds(r0, CH64)], dst_i)

            @pl.loop(0, CH64)
            def _(j):
                srow = src_i.at[j]
                for k in range(ROW // L):
                    v = srow[pl.ds(k * L, L)]
                    srow[pl.ds(k * L, L)] = v + off

            @pl.loop(0, CH64)
            def _(j):
                pltpu.sync_copy(y.at[src_i.at[j]], rows)
                pltpu.sync_copy(rows, accS.at[dst_i.at[j]], add=True)

        plsc.subcore_barrier()

        @pl.loop(0, DUMP)
        def _(k):
            pltpu.sync_copy(accS.at[pl.ds(myacc + k * ROW, ROW)], rows)
            pltpu.sync_copy(rows, acc_out.at[pl.ds(off + myacc + k * ROW, ROW)])

        plsc.subcore_barrier()


@functools.partial(
    pl.kernel,
    out_type=jax.ShapeDtypeStruct((4 * NP, L), jnp.float32),
    mesh=plsc.VectorSubcoreMesh(core_axis_name="c", subcore_axis_name="s"),
    scratch_types=[
        pltpu.VMEM_SHARED((NP, L), jnp.float32),
        pltpu.VMEM((CH64, ROW), jnp.int32),
        pltpu.VMEM((CH64, ROW), jnp.int32),
        pltpu.VMEM((ROW, L), jnp.float32),
        pltpu.VMEM((ROW, L), jnp.float32),
    ],
)
def _sc_edge64(y, srcR, dstR, acc_out, accS, src_i, dst_i, rows, zbuf):
    _edge64_body(y, srcR, dstR, acc_out, accS, src_i, dst_i, rows, zbuf)


def _edge8_body(y, srcR, dstR, acc_out, accS, src_i, dst_i, rows, zbuf):
    c = lax.axis_index("c")
    s = lax.axis_index("s")
    w = c * NS + s
    zeros16 = jnp.zeros((L,), jnp.float32)

    @pl.loop(0, ROW)
    def _(j):
        zbuf[j] = zeros16

    myacc = s * NPT

    @pl.loop(0, DUMP)
    def _(k):
        pltpu.sync_copy(zbuf, accS.at[pl.ds(myacc + k * ROW, ROW)])

    plsc.subcore_barrier()

    rowbase = w * RPT8
    for st in range(ST8):
        r0 = rowbase + st * CH8
        pltpu.sync_copy(srcR.at[pl.ds(r0, CH8)], src_i)
        pltpu.sync_copy(dstR.at[pl.ds(r0, CH8)], dst_i)

        @pl.loop(0, CH8)
        def _(j):
            pltpu.sync_copy(y.at[src_i.at[j]], rows)
            pltpu.sync_copy(rows, accS.at[dst_i.at[j]], add=True)

    plsc.subcore_barrier()

    off = c * NP

    @pl.loop(0, DUMP)
    def _(k):
        pltpu.sync_copy(accS.at[pl.ds(myacc + k * ROW, ROW)], rows)
        pltpu.sync_copy(rows, acc_out.at[pl.ds(off + myacc + k * ROW, ROW)])


@functools.partial(
    pl.kernel,
    out_type=jax.ShapeDtypeStruct((NC * NP, L), jnp.float32),
    mesh=plsc.VectorSubcoreMesh(core_axis_name="c", subcore_axis_name="s"),
    scratch_types=[
        pltpu.VMEM_SHARED((NP, L), jnp.float32),
        pltpu.VMEM((CH8, ROW), jnp.int32),
        pltpu.VMEM((CH8, ROW), jnp.int32),
        pltpu.VMEM((ROW, L), jnp.float32),
        pltpu.VMEM((ROW, L), jnp.float32),
    ],
)
def _sc_edge8(y, srcR, dstR, acc_out, accS, src_i, dst_i, rows, zbuf):
    _edge8_body(y, srcR, dstR, acc_out, accS, src_i, dst_i, rows, zbuf)


def _t0_body(degs_ref, x_ref, w1_ref, dinv_ref, xw_ref, y_ref):
    deg = jnp.sum(degs_ref[...], axis=0) + 1.0
    dinv = lax.rsqrt(deg)
    dinv_ref[...] = dinv[:, None]
    xw = jnp.dot(x_ref[...], w1_ref[...], preferred_element_type=jnp.float32)
    xw_ref[...] = xw
    y = xw * dinv[:, None]
    for f in range(4):
        y_ref[f] = y[:, f * L:(f + 1) * L]


def _tc_t0(degs, xpad, W1):
    return pl.pallas_call(
        _t0_body,
        grid=(GRID,),
        in_specs=[
            pl.BlockSpec((NC * NS, BLK), lambda i: (0, i)),
            pl.BlockSpec((BLK, D_IN), lambda i: (i, 0)),
            pl.BlockSpec((D_IN, D_HID), lambda i: (0, 0)),
        ],
        out_specs=[
            pl.BlockSpec((BLK, 1), lambda i: (i, 0)),
            pl.BlockSpec((BLK, D_HID), lambda i: (i, 0)),
            pl.BlockSpec((4, BLK, L), lambda i: (0, i, 0)),
        ],
        out_shape=[
            jax.ShapeDtypeStruct((NP, 1), jnp.float32),
            jax.ShapeDtypeStruct((NP, D_HID), jnp.float32),
            jax.ShapeDtypeStruct((4, NP, L), jnp.float32),
        ],
    )(degs, xpad, W1)


def _mid_body(acc_ref, xw_ref, dinv_ref, w_ref, b_ref, xwn_ref, y_ref, *, dout):
    dinv = dinv_ref[...]
    agg = jnp.concatenate([acc_ref[f] for f in range(4)], axis=1)
    h = jnp.maximum(agg * dinv + xw_ref[...] * dinv * dinv + b_ref[...], 0.0)
    xwn = jnp.dot(h, w_ref[...], preferred_element_type=jnp.float32)
    xwn_ref[...] = xwn
    y = xwn * dinv
    if dout == D_HID:
        for f in range(4):
            y_ref[f] = y[:, f * L:(f + 1) * L]
    else:
        y_ref[...] = jnp.concatenate(
            [y, jnp.zeros((BLK, L - dout), jnp.float32)], axis=1)


def _tc_mid(acc, xw, dinv, W, b, dout):
    if dout == D_HID:
        y_spec = pl.BlockSpec((4, BLK, L), lambda i: (0, i, 0))
        y_shape = jax.ShapeDtypeStruct((4, NP, L), jnp.float32)
    else:
        y_spec = pl.BlockSpec((BLK, L), lambda i: (i, 0))
        y_shape = jax.ShapeDtypeStruct((NP, L), jnp.float32)
    return pl.pallas_call(
        functools.partial(_mid_body, dout=dout),
        grid=(GRID,),
        in_specs=[
            pl.BlockSpec((4, BLK, L), lambda i: (0, i, 0)),
            pl.BlockSpec((BLK, D_HID), lambda i: (i, 0)),
            pl.BlockSpec((BLK, 1), lambda i: (i, 0)),
            pl.BlockSpec((D_HID, dout), lambda i: (0, 0)),
            pl.BlockSpec((1, D_HID), lambda i: (0, 0)),
        ],
        out_specs=[
            pl.BlockSpec((BLK, dout), lambda i: (i, 0)),
            y_spec,
        ],
        out_shape=[
            jax.ShapeDtypeStruct((NP, dout), jnp.float32),
            y_shape,
        ],
    )(acc, xw, dinv, W, b)


def _t3_body(acc_ref, xw_ref, dinv_ref, b_ref, out_ref):
    dinv = dinv_ref[...]
    agg = acc_ref[0, :, :D_OUT] + acc_ref[1, :, :D_OUT]
    out_ref[...] = agg * dinv + xw_ref[...] * dinv * dinv + b_ref[...]


def _tc_t3(acc2, xw3, dinv, b3):
    return pl.pallas_call(
        _t3_body,
        grid=(GRID,),
        in_specs=[
            pl.BlockSpec((NC, BLK, L), lambda i: (0, i, 0)),
            pl.BlockSpec((BLK, D_OUT), lambda i: (i, 0)),
            pl.BlockSpec((BLK, 1), lambda i: (i, 0)),
            pl.BlockSpec((1, D_OUT), lambda i: (0, 0)),
        ],
        out_specs=pl.BlockSpec((BLK, D_OUT), lambda i: (i, 0)),
        out_shape=jax.ShapeDtypeStruct((NP, D_OUT), jnp.float32),
    )(acc2, xw3, dinv, b3)


def kernel(x, edge_index, W1, b1, W2, b2, W3, b3):
    src = edge_index[0].astype(jnp.int32)
    dst = edge_index[1].astype(jnp.int32)
    padv = jnp.full((EP - E,), PADNODE, jnp.int32)
    srcR = jnp.concatenate([src, padv]).reshape(EROWS, ROW)
    dstR = jnp.concatenate([dst, padv]).reshape(EROWS, ROW)
    xpad = jnp.pad(x, ((0, NP - N), (0, 0)))

    degs = _sc_deg(dstR).reshape(NC * NS, NP)
    dinv, xw1, y1 = _tc_t0(degs, xpad, W1)

    acc1 = _sc_edge64(y1.reshape(4 * NP, L), srcR, dstR).reshape(4, NP, L)
    xw2, y2 = _tc_mid(acc1, xw1, dinv, W2, b1.reshape(1, D_HID), D_HID)

    acc2 = _sc_edge64(y2.reshape(4 * NP, L), srcR, dstR).reshape(4, NP, L)
    xw3, y3 = _tc_mid(acc2, xw2, dinv, W3, b2.reshape(1, D_HID), D_OUT)

    acc3 = _sc_edge8(y3, srcR, dstR).reshape(NC, NP, L)
    out = _tc_t3(acc3, xw3, dinv, b3.reshape(1, D_OUT))
    return out[:N]


# speedup vs baseline: 10.7545x; 10.7545x over previous
import functools

import jax
import jax.numpy as jnp
from jax import lax
from jax.experimental import pallas as pl
from jax.experimental.pallas import tpu as pltpu
from jax.experimental.pallas import tpu_sc as plsc

N = 100000
E = 1600000
D_IN = 16
D_HID = 64
D_OUT = 8

NP = 102400
ROW = 128
EROWS = 12544
EP = EROWS * ROW
PADNODE = NP - 1

NC = 2
NS = 16
L = 16

RPT64 = EROWS // NS
ST64 = 14
CH64 = RPT64 // ST64
RPT8 = EROWS // (NC * NS)
ST8 = 7
CH8 = RPT8 // ST8
NPT = NP // NS
DUMP = NPT // ROW

BLK = 2048
GRID = NP // BLK


def _deg_body(dstR, deg_out, degT, stage):
    c = lax.axis_index("c")
    s = lax.axis_index("s")
    w = c * NS + s
    zeros16 = jnp.zeros((L,), jnp.float32)
    ones16 = jnp.ones((L,), jnp.float32)

    @pl.loop(0, NP // L)
    def _(j):
        degT[pl.ds(j * L, L)] = zeros16

    rowbase = w * RPT8
    for st in range(ST8):
        pltpu.sync_copy(dstR.at[pl.ds(rowbase + st * CH8, CH8)], stage)

        @pl.loop(0, CH8)
        def _(j):
            row = stage.at[j]
            for k in range(ROW // L):
                idx = row[pl.ds(k * L, L)]
                plsc.addupdate_scatter(degT, [idx], ones16)

    pltpu.sync_copy(degT, deg_out.at[pl.ds(w * NP, NP)])


@functools.partial(
    pl.kernel,
    out_type=jax.ShapeDtypeStruct((NC * NS * NP,), jnp.float32),
    mesh=plsc.VectorSubcoreMesh(core_axis_name="c", subcore_axis_name="s"),
    scratch_types=[
        pltpu.VMEM((NP,), jnp.float32),
        pltpu.VMEM((CH8, ROW), jnp.int32),
    ],
    compiler_params=pltpu.CompilerParams(needs_layout_passes=False, use_tc_tiling_on_sc=False),
)
def _sc_deg(dstR, deg_out, degT, stage):
    _deg_body(dstR, deg_out, degT, stage)


def _edge64_body(y, srcR, dstR, acc_out, accS, src_i, dst_i, rows, zbuf):
    c = lax.axis_index("c")
    s = lax.axis_index("s")
    zeros16 = jnp.zeros((L,), jnp.float32)

    @pl.loop(0, ROW)
    def _(j):
        zbuf[j] = zeros16

    rowbase = s * RPT64
    myacc = s * NPT
    for p in range(2):
        chunk = 2 * c + p
        off = chunk * NP

        @pl.loop(0, DUMP)
        def _(k):
            pltpu.sync_copy(zbuf, accS.at[pl.ds(myacc + k * ROW, ROW)])

        plsc.subcore_barrier()

        for st in range(ST64):
            r0 = rowbase + st * CH64
            pltpu.sync_copy(srcR.at[pl.ds(r0, CH64)], src_i)
            pltpu.sync_copy(dstR.at[pl.ds(r0, CH64)], dst_i)

            @pl.loop(0, CH64)
            def _(j):
                srow = src_i.at[j]
                for k in range(ROW // L):
                    v = srow[pl.ds(k * L, L)]
                    srow[pl.ds(k * L, L)] = v + off

            @pl.loop(0, CH64)
            def _(j):
                pltpu.sync_copy(y.at[src_i.at[j]], rows)
                pltpu.sync_copy(rows, accS.at[dst_i.at[j]], add=True)

        plsc.subcore_barrier()

        @pl.loop(0, DUMP)
        def _(k):
            pltpu.sync_copy(accS.at[pl.ds(myacc + k * ROW, ROW)], rows)
            pltpu.sync_copy(rows, acc_out.at[pl.ds(off + myacc + k * ROW, ROW)])

        plsc.subcore_barrier()


@functools.partial(
    pl.kernel,
    out_type=jax.ShapeDtypeStruct((4 * NP, L), jnp.float32),
    mesh=plsc.VectorSubcoreMesh(core_axis_name="c", subcore_axis_name="s"),
    scratch_types=[
        pltpu.VMEM_SHARED((NP, L), jnp.float32),
        pltpu.VMEM((CH64, ROW), jnp.int32),
        pltpu.VMEM((CH64, ROW), jnp.int32),
        pltpu.VMEM((ROW, L), jnp.float32),
        pltpu.VMEM((ROW, L), jnp.float32),
    ],
    compiler_params=pltpu.CompilerParams(needs_layout_passes=False, use_tc_tiling_on_sc=False),
)
def _sc_edge64(y, srcR, dstR, acc_out, accS, src_i, dst_i, rows, zbuf):
    _edge64_body(y, srcR, dstR, acc_out, accS, src_i, dst_i, rows, zbuf)


def _edge8_body(y, srcR, dstR, acc_out, accS, src_i, dst_i, rows, zbuf):
    c = lax.axis_index("c")
    s = lax.axis_index("s")
    w = c * NS + s
    zeros16 = jnp.zeros((L,), jnp.float32)

    @pl.loop(0, ROW)
    def _(j):
        zbuf[j] = zeros16

    myacc = s * NPT

    @pl.loop(0, DUMP)
    def _(k):
        pltpu.sync_copy(zbuf, accS.at[pl.ds(myacc + k * ROW, ROW)])

    plsc.subcore_barrier()

    rowbase = w * RPT8
    for st in range(ST8):
        r0 = rowbase + st * CH8
        pltpu.sync_copy(srcR.at[pl.ds(r0, CH8)], src_i)
        pltpu.sync_copy(dstR.at[pl.ds(r0, CH8)], dst_i)

        @pl.loop(0, CH8)
        def _(j):
            pltpu.sync_copy(y.at[src_i.at[j]], rows)
            pltpu.sync_copy(rows, accS.at[dst_i.at[j]], add=True)

    plsc.subcore_barrier()

    off = c * NP

    @pl.loop(0, DUMP)
    def _(k):
        pltpu.sync_copy(accS.at[pl.ds(myacc + k * ROW, ROW)], rows)
        pltpu.sync_copy(rows, acc_out.at[pl.ds(off + myacc + k * ROW, ROW)])


@functools.partial(
    pl.kernel,
    out_type=jax.ShapeDtypeStruct((NC * NP, L), jnp.float32),
    mesh=plsc.VectorSubcoreMesh(core_axis_name="c", subcore_axis_name="s"),
    scratch_types=[
        pltpu.VMEM_SHARED((NP, L), jnp.float32),
        pltpu.VMEM((CH8, ROW), jnp.int32),
        pltpu.VMEM((CH8, ROW), jnp.int32),
        pltpu.VMEM((ROW, L), jnp.float32),
        pltpu.VMEM((ROW, L), jnp.float32),
    ],
    compiler_params=pltpu.CompilerParams(needs_layout_passes=False, use_tc_tiling_on_sc=False),
)
def _sc_edge8(y, srcR, dstR, acc_out, accS, src_i, dst_i, rows, zbuf):
    _edge8_body(y, srcR, dstR, acc_out, accS, src_i, dst_i, rows, zbuf)


def _t0_body(degs_ref, x_ref, w1_ref, dinv_ref, xw_ref, y_ref):
    deg = jnp.sum(degs_ref[...], axis=0) + 1.0
    dinv = lax.rsqrt(deg)
    dinv_ref[...] = dinv[:, None]
    xw = jnp.dot(x_ref[...], w1_ref[...], preferred_element_type=jnp.float32)
    xw_ref[...] = xw
    y = xw * dinv[:, None]
    for f in range(4):
        y_ref[f] = y[:, f * L:(f + 1) * L]


def _tc_t0(degs, xpad, W1):
    return pl.pallas_call(
        _t0_body,
        grid=(GRID,),
        in_specs=[
            pl.BlockSpec((NC * NS, BLK), lambda i: (0, i)),
            pl.BlockSpec((BLK, D_IN), lambda i: (i, 0)),
            pl.BlockSpec((D_IN, D_HID), lambda i: (0, 0)),
        ],
        out_specs=[
            pl.BlockSpec((BLK, 1), lambda i: (i, 0)),
            pl.BlockSpec((BLK, D_HID), lambda i: (i, 0)),
            pl.BlockSpec((4, BLK, L), lambda i: (0, i, 0)),
        ],
        out_shape=[
            jax.ShapeDtypeStruct((NP, 1), jnp.float32),
            jax.ShapeDtypeStruct((NP, D_HID), jnp.float32),
            jax.ShapeDtypeStruct((4, NP, L), jnp.float32),
        ],
    )(degs, xpad, W1)


def _mid_body(acc_ref, xw_ref, dinv_ref, w_ref, b_ref, xwn_ref, y_ref, *, dout):
    dinv = dinv_ref[...]
    agg = jnp.concatenate([acc_ref[f] for f in range(4)], axis=1)
    h = jnp.maximum(agg * dinv + xw_ref[...] * dinv * dinv + b_ref[...], 0.0)
    xwn = jnp.dot(h, w_ref[...], preferred_element_type=jnp.float32)
    xwn_ref[...] = xwn
    y = xwn * dinv
    if dout == D_HID:
        for f in range(4):
            y_ref[f] = y[:, f * L:(f + 1) * L]
    else:
        y_ref[...] = jnp.concatenate(
            [y, jnp.zeros((BLK, L - dout), jnp.float32)], axis=1)


def _tc_mid(acc, xw, dinv, W, b, dout):
    if dout == D_HID:
        y_spec = pl.BlockSpec((4, BLK, L), lambda i: (0, i, 0))
        y_shape = jax.ShapeDtypeStruct((4, NP, L), jnp.float32)
    else:
        y_spec = pl.BlockSpec((BLK, L), lambda i: (i, 0))
        y_shape = jax.ShapeDtypeStruct((NP, L), jnp.float32)
    return pl.pallas_call(
        functools.partial(_mid_body, dout=dout),
        grid=(GRID,),
        in_specs=[
            pl.BlockSpec((4, BLK, L), lambda i: (0, i, 0)),
            pl.BlockSpec((BLK, D_HID), lambda i: (i, 0)),
            pl.BlockSpec((BLK, 1), lambda i: (i, 0)),
            pl.BlockSpec((D_HID, dout), lambda i: (0, 0)),
            pl.BlockSpec((1, D_HID), lambda i: (0, 0)),
        ],
        out_specs=[
            pl.BlockSpec((BLK, dout), lambda i: (i, 0)),
            y_spec,
        ],
        out_shape=[
            jax.ShapeDtypeStruct((NP, dout), jnp.float32),
            y_shape,
        ],
    )(acc, xw, dinv, W, b)


def _t3_body(acc_ref, xw_ref, dinv_ref, b_ref, out_ref):
    dinv = dinv_ref[...]
    agg = acc_ref[0, :, :D_OUT] + acc_ref[1, :, :D_OUT]
    out_ref[...] = agg * dinv + xw_ref[...] * dinv * dinv + b_ref[...]


def _tc_t3(acc2, xw3, dinv, b3):
    return pl.pallas_call(
        _t3_body,
        grid=(GRID,),
        in_specs=[
            pl.BlockSpec((NC, BLK, L), lambda i: (0, i, 0)),
            pl.BlockSpec((BLK, D_OUT), lambda i: (i, 0)),
            pl.BlockSpec((BLK, 1), lambda i: (i, 0)),
            pl.BlockSpec((1, D_OUT), lambda i: (0, 0)),
        ],
        out_specs=pl.BlockSpec((BLK, D_OUT), lambda i: (i, 0)),
        out_shape=jax.ShapeDtypeStruct((NP, D_OUT), jnp.float32),
    )(acc2, xw3, dinv, b3)


def kernel(x, edge_index, W1, b1, W2, b2, W3, b3):
    src = edge_index[0].astype(jnp.int32)
    dst = edge_index[1].astype(jnp.int32)
    padv = jnp.full((EP - E,), PADNODE, jnp.int32)
    srcR = jnp.concatenate([src, padv]).reshape(EROWS, ROW)
    dstR = jnp.concatenate([dst, padv]).reshape(EROWS, ROW)
    xpad = jnp.pad(x, ((0, NP - N), (0, 0)))

    degs = _sc_deg(dstR).reshape(NC * NS, NP)
    dinv, xw1, y1 = _tc_t0(degs, xpad, W1)

    acc1 = _sc_edge64(y1.reshape(4 * NP, L), srcR, dstR).reshape(4, NP, L)
    xw2, y2 = _tc_mid(acc1, xw1, dinv, W2, b1.reshape(1, D_HID), D_HID)

    acc2 = _sc_edge64(y2.reshape(4 * NP, L), srcR, dstR).reshape(4, NP, L)
    xw3, y3 = _tc_mid(acc2, xw2, dinv, W3, b2.reshape(1, D_HID), D_OUT)

    acc3 = _sc_edge8(y3, srcR, dstR).reshape(NC, NP, L)
    out = _tc_t3(acc3, xw3, dinv, b3.reshape(1, D_OUT))
    return out[:N]


# speedup vs baseline: 17.5533x; 1.6322x over previous
import functools

import jax
import jax.numpy as jnp
from jax import lax
from jax.experimental import pallas as pl
from jax.experimental.pallas import tpu as pltpu
from jax.experimental.pallas import tpu_sc as plsc

N = 100000
E = 1600000
D_IN = 16
D_HID = 64
D_OUT = 8

NP = 102400
ROW = 128
EROWS = 12544
EP = EROWS * ROW
PADNODE = NP - 1

NC = 2
NS = 16
L = 16

WAVE = 4
NBUF = 2 * WAVE
RPT64 = EROWS // NS
CH = 28
ST64 = RPT64 // CH
NWAVES = CH // WAVE
RPT8 = EROWS // (NC * NS)
ST8 = RPT8 // CH
NPT = NP // NS
DUMP = NPT // ROW

BLK = 2048
GRID = NP // BLK

_SC_PARAMS = pltpu.CompilerParams(
    needs_layout_passes=False, use_tc_tiling_on_sc=False)


def _deg_body(dstR, deg_out, degT, stage):
    c = lax.axis_index("c")
    s = lax.axis_index("s")
    w = c * NS + s
    zeros16 = jnp.zeros((L,), jnp.float32)
    ones16 = jnp.ones((L,), jnp.float32)

    @pl.loop(0, NP // L)
    def _(j):
        degT[pl.ds(j * L, L)] = zeros16

    rowbase = w * RPT8
    for st in range(RPT8 // 56):
        pltpu.sync_copy(dstR.at[pl.ds(rowbase + st * 56, 56)], stage)

        @pl.loop(0, 56)
        def _(j):
            row = stage.at[j]
            for k in range(ROW // L):
                idx = row[pl.ds(k * L, L)]
                plsc.addupdate_scatter(degT, [idx], ones16)

    pltpu.sync_copy(degT, deg_out.at[pl.ds(w * NP, NP)])


@functools.partial(
    pl.kernel,
    out_type=jax.ShapeDtypeStruct((NC * NS * NP,), jnp.float32),
    mesh=plsc.VectorSubcoreMesh(core_axis_name="c", subcore_axis_name="s"),
    scratch_types=[
        pltpu.VMEM((NP,), jnp.float32),
        pltpu.VMEM((56, ROW), jnp.int32),
    ],
    compiler_params=_SC_PARAMS,
)
def _sc_deg(dstR, deg_out, degT, stage):
    _deg_body(dstR, deg_out, degT, stage)


def _sweep(ytab, srcR, dstR, accS, src_i, dst_i, bufs, semG, semS,
           rowbase, nstages):
    @pl.loop(0, nstages)
    def _(st):
        r0 = rowbase + st * CH
        pltpu.sync_copy(srcR.at[pl.ds(r0, CH)], src_i)
        pltpu.sync_copy(dstR.at[pl.ds(r0, CH)], dst_i)

        gd = {}
        sd = {}
        gd[0] = [
            pltpu.async_copy(ytab.at[src_i.at[b]], bufs[b], semG)
            for b in range(WAVE)
        ]
        for w in range(NWAVES):
            grp = [bufs[(w % 2) * WAVE + b] for b in range(WAVE)]
            nxt = [bufs[((w + 1) % 2) * WAVE + b] for b in range(WAVE)]
            for d in gd[w]:
                d.wait()
            if w >= 1:
                for d in sd[w - 1]:
                    d.wait()
            if w + 1 < NWAVES:
                gd[w + 1] = [
                    pltpu.async_copy(
                        ytab.at[src_i.at[(w + 1) * WAVE + b]], nxt[b], semG)
                    for b in range(WAVE)
                ]
            sd[w] = [
                pltpu.async_copy(
                    grp[b], accS.at[dst_i.at[w * WAVE + b]], semS, add=True)
                for b in range(WAVE)
            ]
        for d in sd[NWAVES - 1]:
            d.wait()


def _zero_acc(accS, zbuf, myacc):
    zeros16 = jnp.zeros((L,), jnp.float32)

    @pl.loop(0, ROW)
    def _(j):
        zbuf[j] = zeros16

    @pl.loop(0, DUMP)
    def _(k):
        pltpu.sync_copy(zbuf, accS.at[pl.ds(myacc + k * ROW, ROW)])


def _dump_acc(accS, buf, acc_out, myacc, out_base):
    @pl.loop(0, DUMP)
    def _(k):
        pltpu.sync_copy(accS.at[pl.ds(myacc + k * ROW, ROW)], buf)
        pltpu.sync_copy(buf, acc_out.at[pl.ds(out_base + myacc + k * ROW, ROW)])


def _edge64_body(y0, y1, y2, y3, srcR, dstR, acc_out, accS, src_i, dst_i,
                 *rest):
    bufs = rest[:NBUF]
    semG, semS = rest[NBUF], rest[NBUF + 1]
    c = lax.axis_index("c")
    s = lax.axis_index("s")
    rowbase = s * RPT64
    myacc = s * NPT

    def one_pass(ytab, out_base):
        _zero_acc(accS, bufs[0], myacc)
        plsc.subcore_barrier()
        _sweep(ytab, srcR, dstR, accS, src_i, dst_i, bufs, semG, semS,
               rowbase, ST64)
        plsc.subcore_barrier()
        _dump_acc(accS, bufs[0], acc_out, myacc, out_base)
        plsc.subcore_barrier()

    @pl.when(c == 0)
    def _():
        one_pass(y0, 0)
        one_pass(y1, NP)

    @pl.when(c == 1)
    def _():
        one_pass(y2, 2 * NP)
        one_pass(y3, 3 * NP)


@functools.partial(
    pl.kernel,
    out_type=jax.ShapeDtypeStruct((4 * NP, L), jnp.float32),
    mesh=plsc.VectorSubcoreMesh(core_axis_name="c", subcore_axis_name="s"),
    scratch_types=[
        pltpu.VMEM_SHARED((NP, L), jnp.float32),
        pltpu.VMEM((CH, ROW), jnp.int32),
        pltpu.VMEM((CH, ROW), jnp.int32),
    ] + [pltpu.VMEM((ROW, L), jnp.float32) for _ in range(NBUF)] + [
        pltpu.SemaphoreType.DMA,
        pltpu.SemaphoreType.DMA,
    ],
    compiler_params=_SC_PARAMS,
)
def _sc_edge64(y0, y1, y2, y3, srcR, dstR, acc_out, accS, src_i, dst_i,
               *rest):
    _edge64_body(y0, y1, y2, y3, srcR, dstR, acc_out, accS, src_i, dst_i,
                 *rest)


def _edge8_body(y, srcR, dstR, acc_out, accS, src_i, dst_i, *rest):
    bufs = rest[:NBUF]
    semG, semS = rest[NBUF], rest[NBUF + 1]
    c = lax.axis_index("c")
    s = lax.axis_index("s")
    w = c * NS + s
    myacc = s * NPT

    _zero_acc(accS, bufs[0], myacc)
    plsc.subcore_barrier()
    _sweep(y, srcR, dstR, accS, src_i, dst_i, bufs, semG, semS,
           w * RPT8, ST8)
    plsc.subcore_barrier()
    _dump_acc(accS, bufs[0], acc_out, myacc, c * NP)


@functools.partial(
    pl.kernel,
    out_type=jax.ShapeDtypeStruct((NC * NP, L), jnp.float32),
    mesh=plsc.VectorSubcoreMesh(core_axis_name="c", subcore_axis_name="s"),
    scratch_types=[
        pltpu.VMEM_SHARED((NP, L), jnp.float32),
        pltpu.VMEM((CH, ROW), jnp.int32),
        pltpu.VMEM((CH, ROW), jnp.int32),
    ] + [pltpu.VMEM((ROW, L), jnp.float32) for _ in range(NBUF)] + [
        pltpu.SemaphoreType.DMA,
        pltpu.SemaphoreType.DMA,
    ],
    compiler_params=_SC_PARAMS,
)
def _sc_edge8(y, srcR, dstR, acc_out, accS, src_i, dst_i, *rest):
    _edge8_body(y, srcR, dstR, acc_out, accS, src_i, dst_i, *rest)


def _t0_body(degs_ref, x_ref, w1_ref, dinv_ref, xw_ref, *y_refs):
    deg = jnp.sum(degs_ref[...], axis=0) + 1.0
    dinv = lax.rsqrt(deg)
    dinv_ref[...] = dinv[:, None]
    xw = jnp.dot(x_ref[...], w1_ref[...], preferred_element_type=jnp.float32)
    xw_ref[...] = xw
    y = xw * dinv[:, None]
    for f in range(4):
        y_refs[f][...] = y[:, f * L:(f + 1) * L]


def _tc_t0(degs, xpad, W1):
    return pl.pallas_call(
        _t0_body,
        grid=(GRID,),
        in_specs=[
            pl.BlockSpec((NC * NS, BLK), lambda i: (0, i)),
            pl.BlockSpec((BLK, D_IN), lambda i: (i, 0)),
            pl.BlockSpec((D_IN, D_HID), lambda i: (0, 0)),
        ],
        out_specs=[
            pl.BlockSpec((BLK, 1), lambda i: (i, 0)),
            pl.BlockSpec((BLK, D_HID), lambda i: (i, 0)),
        ] + [pl.BlockSpec((BLK, L), lambda i: (i, 0)) for _ in range(4)],
        out_shape=[
            jax.ShapeDtypeStruct((NP, 1), jnp.float32),
            jax.ShapeDtypeStruct((NP, D_HID), jnp.float32),
        ] + [jax.ShapeDtypeStruct((NP, L), jnp.float32) for _ in range(4)],
    )(degs, xpad, W1)


def _mid_body(acc_ref, xw_ref, dinv_ref, w_ref, b_ref, xwn_ref, *y_refs,
              dout):
    dinv = dinv_ref[...]
    agg = jnp.concatenate([acc_ref[f] for f in range(4)], axis=1)
    h = jnp.maximum(agg * dinv + xw_ref[...] * dinv * dinv + b_ref[...], 0.0)
    xwn = jnp.dot(h, w_ref[...], preferred_element_type=jnp.float32)
    xwn_ref[...] = xwn
    y = xwn * dinv
    if dout == D_HID:
        for f in range(4):
            y_refs[f][...] = y[:, f * L:(f + 1) * L]
    else:
        y_refs[0][...] = jnp.concatenate(
            [y, jnp.zeros((BLK, L - dout), jnp.float32)], axis=1)


def _tc_mid(acc, xw, dinv, W, b, dout):
    n_y = 4 if dout == D_HID else 1
    return pl.pallas_call(
        functools.partial(_mid_body, dout=dout),
        grid=(GRID,),
        in_specs=[
            pl.BlockSpec((4, BLK, L), lambda i: (0, i, 0)),
            pl.BlockSpec((BLK, D_HID), lambda i: (i, 0)),
            pl.BlockSpec((BLK, 1), lambda i: (i, 0)),
            pl.BlockSpec((D_HID, dout), lambda i: (0, 0)),
            pl.BlockSpec((1, D_HID), lambda i: (0, 0)),
        ],
        out_specs=[
            pl.BlockSpec((BLK, dout), lambda i: (i, 0)),
        ] + [pl.BlockSpec((BLK, L), lambda i: (i, 0)) for _ in range(n_y)],
        out_shape=[
            jax.ShapeDtypeStruct((NP, dout), jnp.float32),
        ] + [jax.ShapeDtypeStruct((NP, L), jnp.float32) for _ in range(n_y)],
    )(acc, xw, dinv, W, b)


def _t3_body(acc_ref, xw_ref, dinv_ref, b_ref, out_ref):
    dinv = dinv_ref[...]
    agg = acc_ref[0, :, :D_OUT] + acc_ref[1, :, :D_OUT]
    out_ref[...] = agg * dinv + xw_ref[...] * dinv * dinv + b_ref[...]


def _tc_t3(acc2, xw3, dinv, b3):
    return pl.pallas_call(
        _t3_body,
        grid=(GRID,),
        in_specs=[
            pl.BlockSpec((NC, BLK, L), lambda i: (0, i, 0)),
            pl.BlockSpec((BLK, D_OUT), lambda i: (i, 0)),
            pl.BlockSpec((BLK, 1), lambda i: (i, 0)),
            pl.BlockSpec((1, D_OUT), lambda i: (0, 0)),
        ],
        out_specs=pl.BlockSpec((BLK, D_OUT), lambda i: (i, 0)),
        out_shape=jax.ShapeDtypeStruct((NP, D_OUT), jnp.float32),
    )(acc2, xw3, dinv, b3)


def kernel(x, edge_index, W1, b1, W2, b2, W3, b3):
    src = edge_index[0].astype(jnp.int32)
    dst = edge_index[1].astype(jnp.int32)
    padv = jnp.full((EP - E,), PADNODE, jnp.int32)
    srcR = jnp.concatenate([src, padv]).reshape(EROWS, ROW)
    dstR = jnp.concatenate([dst, padv]).reshape(EROWS, ROW)
    xpad = jnp.pad(x, ((0, NP - N), (0, 0)))

    degs = _sc_deg(dstR).reshape(NC * NS, NP)
    dinv, xw1, *y1 = _tc_t0(degs, xpad, W1)

    acc1 = _sc_edge64(*y1, srcR, dstR).reshape(4, NP, L)
    xw2, *y2 = _tc_mid(acc1, xw1, dinv, W2, b1.reshape(1, D_HID), D_HID)

    acc2 = _sc_edge64(*y2, srcR, dstR).reshape(4, NP, L)
    xw3, y3 = _tc_mid(acc2, xw2, dinv, W3, b2.reshape(1, D_HID), D_OUT)

    acc3 = _sc_edge8(y3, srcR, dstR).reshape(NC, NP, L)
    out = _tc_t3(acc3, xw3, dinv, b3.reshape(1, D_OUT))
    return out[:N]


# speedup vs baseline: 22.7317x; 1.2950x over previous
import functools

import jax
import jax.numpy as jnp
from jax import lax
from jax.experimental import pallas as pl
from jax.experimental.pallas import tpu as pltpu
from jax.experimental.pallas import tpu_sc as plsc

N = 100000
E = 1600000
D_IN = 16
D_HID = 64
D_OUT = 8

NP = 102400
ROW = 128
EROWS = 12800
EP = EROWS * ROW
NPAD = NP - N

NC = 2
NS = 16
L = 16

WAVE = 4
NBUF = 2 * WAVE
RPT64 = EROWS // NS
CH = 40
ST64 = RPT64 // CH
NWAVES = CH // WAVE
RPT8 = EROWS // (NC * NS)
ST8 = RPT8 // CH
NPT = NP // NS
DUMP = NPT // ROW
DCH = 50
DST = RPT8 // DCH

BLK = 2048
GRID = (N + BLK - 1) // BLK
BR = BLK // ROW

_SC_PARAMS = pltpu.CompilerParams(
    needs_layout_passes=False, use_tc_tiling_on_sc=False)


def _deg_body(dstR, deg_out, degT, stage):
    c = lax.axis_index("c")
    s = lax.axis_index("s")
    w = c * NS + s
    zeros16 = jnp.zeros((L,), jnp.float32)
    ones16 = jnp.ones((L,), jnp.float32)

    @pl.loop(0, NP // ROW)
    def _(j):
        for k in range(ROW // L):
            degT[j, pl.ds(k * L, L)] = zeros16

    rowbase = w * RPT8
    for st in range(DST):
        pltpu.sync_copy(dstR.at[pl.ds(rowbase + st * DCH, DCH)], stage)

        @pl.loop(0, DCH)
        def _(j):
            row = stage.at[j]
            for k in range(ROW // L):
                idx = row[pl.ds(k * L, L)]
                plsc.addupdate_scatter(
                    degT,
                    [lax.shift_right_logical(idx, 7),
                     lax.bitwise_and(idx, 127)],
                    ones16)

    pltpu.sync_copy(degT, deg_out.at[pl.ds(w * (NP // ROW), NP // ROW)])


@functools.partial(
    pl.kernel,
    out_type=jax.ShapeDtypeStruct((NC * NS * (NP // ROW), ROW), jnp.float32),
    mesh=plsc.VectorSubcoreMesh(core_axis_name="c", subcore_axis_name="s"),
    scratch_types=[
        pltpu.VMEM((NP // ROW, ROW), jnp.float32),
        pltpu.VMEM((DCH, ROW), jnp.int32),
    ],
    compiler_params=_SC_PARAMS,
)
def _sc_deg(dstR, deg_out, degT, stage):
    _deg_body(dstR, deg_out, degT, stage)


def _sweep(ytab, srcR, dstR, accS, src_i, dst_i, bufs, semG, semS,
           rowbase, nstages):
    @pl.loop(0, nstages)
    def _(st):
        r0 = rowbase + st * CH
        pltpu.sync_copy(srcR.at[pl.ds(r0, CH)], src_i)
        pltpu.sync_copy(dstR.at[pl.ds(r0, CH)], dst_i)

        gd = {}
        sd = {}
        gd[0] = [
            pltpu.async_copy(ytab.at[src_i.at[b]], bufs[b], semG)
            for b in range(WAVE)
        ]
        for w in range(NWAVES):
            grp = [bufs[(w % 2) * WAVE + b] for b in range(WAVE)]
            nxt = [bufs[((w + 1) % 2) * WAVE + b] for b in range(WAVE)]
            for d in gd[w]:
                d.wait()
            if w >= 1:
                for d in sd[w - 1]:
                    d.wait()
            if w + 1 < NWAVES:
                gd[w + 1] = [
                    pltpu.async_copy(
                        ytab.at[src_i.at[(w + 1) * WAVE + b]], nxt[b], semG)
                    for b in range(WAVE)
                ]
            sd[w] = [
                pltpu.async_copy(
                    grp[b], accS.at[dst_i.at[w * WAVE + b]], semS, add=True)
                for b in range(WAVE)
            ]
        for d in sd[NWAVES - 1]:
            d.wait()


def _zero_acc(accS, zbuf, myacc):
    zeros16 = jnp.zeros((L,), jnp.float32)

    @pl.loop(0, ROW)
    def _(j):
        zbuf[j] = zeros16

    @pl.loop(0, DUMP)
    def _(k):
        pltpu.sync_copy(zbuf, accS.at[pl.ds(myacc + k * ROW, ROW)])


def _dump_acc(accS, buf, acc_out, myacc, out_base):
    @pl.loop(0, DUMP)
    def _(k):
        pltpu.sync_copy(accS.at[pl.ds(myacc + k * ROW, ROW)], buf)
        pltpu.sync_copy(buf, acc_out.at[pl.ds(out_base + myacc + k * ROW, ROW)])


def _edge64_body(y0, y1, y2, y3, srcR, dstR, acc_out, accS, src_i, dst_i,
                 *rest):
    bufs = rest[:NBUF]
    semG, semS = rest[NBUF], rest[NBUF + 1]
    c = lax.axis_index("c")
    s = lax.axis_index("s")
    rowbase = s * RPT64
    myacc = s * NPT

    def one_pass(ytab, out_base):
        _zero_acc(accS, bufs[0], myacc)
        plsc.subcore_barrier()
        _sweep(ytab, srcR, dstR, accS, src_i, dst_i, bufs, semG, semS,
               rowbase, ST64)
        plsc.subcore_barrier()
        _dump_acc(accS, bufs[0], acc_out, myacc, out_base)
        plsc.subcore_barrier()

    @pl.when(c == 0)
    def _():
        one_pass(y0, 0)
        one_pass(y1, NP)

    @pl.when(c == 1)
    def _():
        one_pass(y2, 2 * NP)
        one_pass(y3, 3 * NP)


@functools.partial(
    pl.kernel,
    out_type=jax.ShapeDtypeStruct((4 * NP, L), jnp.float32),
    mesh=plsc.VectorSubcoreMesh(core_axis_name="c", subcore_axis_name="s"),
    scratch_types=[
        pltpu.VMEM_SHARED((NP, L), jnp.float32),
        pltpu.VMEM((CH, ROW), jnp.int32),
        pltpu.VMEM((CH, ROW), jnp.int32),
    ] + [pltpu.VMEM((ROW, L), jnp.float32) for _ in range(NBUF)] + [
        pltpu.SemaphoreType.DMA,
        pltpu.SemaphoreType.DMA,
    ],
    compiler_params=_SC_PARAMS,
)
def _sc_edge64(y0, y1, y2, y3, srcR, dstR, acc_out, accS, src_i, dst_i,
               *rest):
    _edge64_body(y0, y1, y2, y3, srcR, dstR, acc_out, accS, src_i, dst_i,
                 *rest)


def _edge8_body(y, srcR, dstR, acc_out, accS, src_i, dst_i, *rest):
    bufs = rest[:NBUF]
    semG, semS = rest[NBUF], rest[NBUF + 1]
    c = lax.axis_index("c")
    s = lax.axis_index("s")
    w = c * NS + s
    myacc = s * NPT

    _zero_acc(accS, bufs[0], myacc)
    plsc.subcore_barrier()
    _sweep(y, srcR, dstR, accS, src_i, dst_i, bufs, semG, semS,
           w * RPT8, ST8)
    plsc.subcore_barrier()
    _dump_acc(accS, bufs[0], acc_out, myacc, c * NP)


@functools.partial(
    pl.kernel,
    out_type=jax.ShapeDtypeStruct((NC * NP, L), jnp.float32),
    mesh=plsc.VectorSubcoreMesh(core_axis_name="c", subcore_axis_name="s"),
    scratch_types=[
        pltpu.VMEM_SHARED((NP, L), jnp.float32),
        pltpu.VMEM((CH, ROW), jnp.int32),
        pltpu.VMEM((CH, ROW), jnp.int32),
    ] + [pltpu.VMEM((ROW, L), jnp.float32) for _ in range(NBUF)] + [
        pltpu.SemaphoreType.DMA,
        pltpu.SemaphoreType.DMA,
    ],
    compiler_params=_SC_PARAMS,
)
def _sc_edge8(y, srcR, dstR, acc_out, accS, src_i, dst_i, *rest):
    _edge8_body(y, srcR, dstR, acc_out, accS, src_i, dst_i, *rest)


def _hop(v, mid, out):
    return (v.reshape(mid) + 0.0).reshape(out)


def _t0_body(x_ref, w1_ref, *refs):
    deg_refs = refs[:NC * NS]
    dinv_ref, xw_ref = refs[NC * NS], refs[NC * NS + 1]
    y_refs = refs[NC * NS + 2:]
    deg = deg_refs[0][...]
    for r in deg_refs[1:]:
        deg = deg + r[...]
    dinv2d = lax.rsqrt(deg + 1.0)
    dinv_ref[...] = dinv2d
    dinv = _hop(dinv2d, (BR, ROW, 1), (BLK, 1))
    xw = jnp.dot(x_ref[...], w1_ref[...], preferred_element_type=jnp.float32)
    xw_ref[...] = _hop(xw, (BLK // 2, 2, D_HID), (BLK * D_HID // ROW, ROW))
    y = xw * dinv
    for f in range(4):
        y_refs[f][...] = _hop(y[:, f * L:(f + 1) * L],
                              (BLK * L // ROW, 8, L), (BLK * L // ROW, ROW))


def _tc_t0(x, W1, degs):
    return pl.pallas_call(
        _t0_body,
        grid=(GRID,),
        in_specs=[
            pl.BlockSpec((BLK, D_IN), lambda i: (i, 0)),
            pl.BlockSpec((D_IN, D_HID), lambda i: (0, 0)),
        ] + [
            pl.BlockSpec((BR, ROW), lambda i, w=w: (w * (NP // ROW // BR) + i, 0))
            for w in range(NC * NS)
        ],
        out_specs=[
            pl.BlockSpec((BR, ROW), lambda i: (i, 0)),
            pl.BlockSpec((BLK * D_HID // ROW, ROW), lambda i: (i, 0)),
        ] + [pl.BlockSpec((BLK * L // ROW, ROW), lambda i: (i, 0))
             for _ in range(4)],
        out_shape=[
            jax.ShapeDtypeStruct((NP // ROW, ROW), jnp.float32),
            jax.ShapeDtypeStruct((NP * D_HID // ROW, ROW), jnp.float32),
        ] + [jax.ShapeDtypeStruct((NP * L // ROW, ROW), jnp.float32)
             for _ in range(4)],
    )(x, W1, *degs)


def _mid_body(a0, a1, a2, a3, xw_ref, dinv_ref, w_ref, b_ref, xwn_ref,
              *y_refs, dout):
    dinv = _hop(dinv_ref[...], (BR, ROW, 1), (BLK, 1))
    agg = jnp.concatenate(
        [_hop(a[...], (BLK * L // ROW, 8, L), (BLK, L))
         for a in (a0, a1, a2, a3)], axis=1)
    xw = _hop(xw_ref[...], (BLK // 2, 2, D_HID), (BLK, D_HID))
    h = jnp.maximum(agg * dinv + xw * dinv * dinv + b_ref[...], 0.0)
    xwn = jnp.dot(h, w_ref[...], preferred_element_type=jnp.float32)
    if dout == D_HID:
        xwn_ref[...] = _hop(xwn, (BLK // 2, 2, D_HID),
                            (BLK * dout // ROW, ROW))
    else:
        xwn_ref[...] = _hop(xwn, (BLK * dout // ROW, ROW // dout, dout),
                            (BLK * dout // ROW, ROW))
    y = xwn * dinv
    if dout == D_HID:
        for f in range(4):
            y_refs[f][...] = _hop(y[:, f * L:(f + 1) * L],
                                  (BLK * L // ROW, 8, L),
                                  (BLK * L // ROW, ROW))
    else:
        y_refs[0][...] = _hop(
            jnp.concatenate([y, jnp.zeros((BLK, L - dout), jnp.float32)],
                            axis=1),
            (BLK * L // ROW, 8, L), (BLK * L // ROW, ROW))


def _tc_mid(acc, xw, dinv, W, b, dout):
    n_y = 4 if dout == D_HID else 1
    return pl.pallas_call(
        functools.partial(_mid_body, dout=dout),
        grid=(GRID,),
        in_specs=[
            pl.BlockSpec((BLK * L // ROW, ROW),
                         lambda i, f=f: (f * (NP * L // ROW // (BLK * L // ROW)) + i, 0))
            for f in range(4)
        ] + [
            pl.BlockSpec((BLK * D_HID // ROW, ROW), lambda i: (i, 0)),
            pl.BlockSpec((BR, ROW), lambda i: (i, 0)),
            pl.BlockSpec((D_HID, dout), lambda i: (0, 0)),
            pl.BlockSpec((1, D_HID), lambda i: (0, 0)),
        ],
        out_specs=[
            pl.BlockSpec((BLK * dout // ROW, ROW), lambda i: (i, 0)),
        ] + [pl.BlockSpec((BLK * L // ROW, ROW), lambda i: (i, 0))
             for _ in range(n_y)],
        out_shape=[
            jax.ShapeDtypeStruct((NP * dout // ROW, ROW), jnp.float32),
        ] + [jax.ShapeDtypeStruct((NP * L // ROW, ROW), jnp.float32)
             for _ in range(n_y)],
    )(acc, acc, acc, acc, xw, dinv, W, b)


def _t3_body(a0, a1, xw_ref, dinv_ref, b_ref, out_ref):
    dinv = _hop(dinv_ref[...], (BR, ROW, 1), (BLK, 1))
    agg = (_hop(a0[...], (BLK * L // ROW, 8, L), (BLK, L))[:, :D_OUT]
           + _hop(a1[...], (BLK * L // ROW, 8, L), (BLK, L))[:, :D_OUT])
    xw = _hop(xw_ref[...], (BLK * D_OUT // ROW, ROW // D_OUT, D_OUT),
              (BLK, D_OUT))
    out_ref[...] = agg * dinv + xw * dinv * dinv + b_ref[...]


def _tc_t3(acc2, xw3, dinv, b3):
    return pl.pallas_call(
        _t3_body,
        grid=(GRID,),
        in_specs=[
            pl.BlockSpec((BLK * L // ROW, ROW),
                         lambda i, c=c: (c * (NP * L // ROW // (BLK * L // ROW)) + i, 0))
            for c in range(NC)
        ] + [
            pl.BlockSpec((BLK * D_OUT // ROW, ROW), lambda i: (i, 0)),
            pl.BlockSpec((BR, ROW), lambda i: (i, 0)),
            pl.BlockSpec((1, D_OUT), lambda i: (0, 0)),
        ],
        out_specs=pl.BlockSpec((BLK, D_OUT), lambda i: (i, 0)),
        out_shape=jax.ShapeDtypeStruct((N, D_OUT), jnp.float32),
    )(acc2, acc2, xw3, dinv, b3)


def kernel(x, edge_index, W1, b1, W2, b2, W3, b3):
    src = edge_index[0].astype(jnp.int32)
    dst = edge_index[1].astype(jnp.int32)
    padv = N + jnp.arange(EP - E, dtype=jnp.int32) % NPAD
    srcR = jnp.concatenate([src, padv]).reshape(EROWS, ROW)
    dstR = jnp.concatenate([dst, padv]).reshape(EROWS, ROW)

    degs128 = _sc_deg(dstR)
    dinv, xw1, *y1 = _tc_t0(x, W1, [degs128] * (NC * NS))

    y1t = [yc.reshape(NP, L) for yc in y1]
    acc1 = _sc_edge64(*y1t, srcR, dstR).reshape(NP * 4 * L // ROW, ROW)
    xw2, *y2 = _tc_mid(acc1, xw1, dinv, W2, b1.reshape(1, D_HID), D_HID)

    y2t = [yc.reshape(NP, L) for yc in y2]
    acc2 = _sc_edge64(*y2t, srcR, dstR).reshape(NP * 4 * L // ROW, ROW)
    xw3, y3 = _tc_mid(acc2, xw2, dinv, W3, b2.reshape(1, D_HID), D_OUT)

    acc3 = _sc_edge8(y3.reshape(NP, L), srcR, dstR).reshape(
        NP * NC * L // ROW, ROW)
    out = _tc_t3(acc3, xw3, dinv, b3.reshape(1, D_OUT))
    return out


# speedup vs baseline: 27.5930x; 1.2139x over previous
import functools

import jax
import jax.numpy as jnp
from jax import lax
from jax.experimental import pallas as pl
from jax.experimental.pallas import tpu as pltpu
from jax.experimental.pallas import tpu_sc as plsc

N = 100000
E = 1600000
D_IN = 16
D_HID = 64
D_OUT = 8

NP = 102400
ROW = 128
EROWS = 12800
EP = EROWS * ROW
NPAD = NP - N

NC = 2
NS = 16
L = 16

WAVE = 4
NBUF = 2 * WAVE
RPT64 = EROWS // NS
CH = 40
ST64 = RPT64 // CH
NWAVES = CH // WAVE
RPT8 = EROWS // (NC * NS)
ST8 = RPT8 // CH
NPT = NP // NS
DUMP = NPT // ROW
DCH = 50
DST = RPT8 // DCH

BLK = 2048
GRID = (N + BLK - 1) // BLK
BR = BLK // ROW

_SC_PARAMS = pltpu.CompilerParams(
    needs_layout_passes=False, use_tc_tiling_on_sc=False)


def _deg_body(dstR, deg_out, degT, stage):
    c = lax.axis_index("c")
    s = lax.axis_index("s")
    w = c * NS + s
    zeros16 = jnp.zeros((L,), jnp.float32)
    ones16 = jnp.ones((L,), jnp.float32)

    @pl.loop(0, NP // ROW)
    def _(j):
        for k in range(ROW // L):
            degT[j, pl.ds(k * L, L)] = zeros16

    rowbase = w * RPT8
    for st in range(DST):
        pltpu.sync_copy(dstR.at[pl.ds(rowbase + st * DCH, DCH)], stage)

        @pl.loop(0, DCH)
        def _(j):
            row = stage.at[j]
            for k in range(ROW // L):
                idx = row[pl.ds(k * L, L)]
                plsc.addupdate_scatter(
                    degT,
                    [lax.shift_right_logical(idx, 7),
                     lax.bitwise_and(idx, 127)],
                    ones16)

    pltpu.sync_copy(degT, deg_out.at[pl.ds(w * (NP // ROW), NP // ROW)])


@functools.partial(
    pl.kernel,
    out_type=jax.ShapeDtypeStruct((NC * NS * (NP // ROW), ROW), jnp.float32),
    mesh=plsc.VectorSubcoreMesh(core_axis_name="c", subcore_axis_name="s"),
    scratch_types=[
        pltpu.VMEM((NP // ROW, ROW), jnp.float32),
        pltpu.VMEM((DCH, ROW), jnp.int32),
    ],
    compiler_params=_SC_PARAMS,
)
def _sc_deg(dstR, deg_out, degT, stage):
    _deg_body(dstR, deg_out, degT, stage)


def _sweep(ytab, srcR, dstR, accS, src_i, dst_i, bufs, semG, semS,
           rowbase, nstages):
    @pl.loop(0, nstages)
    def _(st):
        r0 = rowbase + st * CH
        pltpu.sync_copy(srcR.at[pl.ds(r0, CH)], src_i)
        pltpu.sync_copy(dstR.at[pl.ds(r0, CH)], dst_i)

        gd = {}
        sd = {}
        gd[0] = [
            pltpu.async_copy(ytab.at[src_i.at[b]], bufs[b], semG)
            for b in range(WAVE)
        ]
        for w in range(NWAVES):
            grp = [bufs[(w % 2) * WAVE + b] for b in range(WAVE)]
            nxt = [bufs[((w + 1) % 2) * WAVE + b] for b in range(WAVE)]
            for d in gd[w]:
                d.wait()
            if w >= 1:
                for d in sd[w - 1]:
                    d.wait()
            if w + 1 < NWAVES:
                gd[w + 1] = [
                    pltpu.async_copy(
                        ytab.at[src_i.at[(w + 1) * WAVE + b]], nxt[b], semG)
                    for b in range(WAVE)
                ]
            sd[w] = [
                pltpu.async_copy(
                    grp[b], accS.at[dst_i.at[w * WAVE + b]], semS, add=True)
                for b in range(WAVE)
            ]
        for d in sd[NWAVES - 1]:
            d.wait()


def _zero_acc(accS, zbuf, myacc):
    zeros16 = jnp.zeros((L,), jnp.float32)

    @pl.loop(0, ROW)
    def _(j):
        zbuf[j] = zeros16

    @pl.loop(0, DUMP)
    def _(k):
        pltpu.sync_copy(zbuf, accS.at[pl.ds(myacc + k * ROW, ROW)])


def _dump_acc(accS, buf, acc_out, myacc, out_base):
    @pl.loop(0, DUMP)
    def _(k):
        pltpu.sync_copy(accS.at[pl.ds(myacc + k * ROW, ROW)], buf)
        pltpu.sync_copy(buf, acc_out.at[pl.ds(out_base + myacc + k * ROW, ROW)])


def _edge64_body(y0, y1, y2, y3, srcR, dstR, acc_out, accS, src_i, dst_i,
                 *rest):
    bufs = rest[:NBUF]
    semG, semS = rest[NBUF], rest[NBUF + 1]
    c = lax.axis_index("c")
    s = lax.axis_index("s")
    rowbase = s * RPT64
    myacc = s * NPT

    def one_pass(ytab, out_base):
        _zero_acc(accS, bufs[0], myacc)
        plsc.subcore_barrier()
        _sweep(ytab, srcR, dstR, accS, src_i, dst_i, bufs, semG, semS,
               rowbase, ST64)
        plsc.subcore_barrier()
        _dump_acc(accS, bufs[0], acc_out, myacc, out_base)
        plsc.subcore_barrier()

    @pl.when(c == 0)
    def _():
        one_pass(y0, 0)
        one_pass(y1, NP)

    @pl.when(c == 1)
    def _():
        one_pass(y2, 2 * NP)
        one_pass(y3, 3 * NP)


@functools.partial(
    pl.kernel,
    out_type=jax.ShapeDtypeStruct((4 * NP, L), jnp.float32),
    mesh=plsc.VectorSubcoreMesh(core_axis_name="c", subcore_axis_name="s"),
    scratch_types=[
        pltpu.VMEM_SHARED((NP, L), jnp.float32),
        pltpu.VMEM((CH, ROW), jnp.int32),
        pltpu.VMEM((CH, ROW), jnp.int32),
    ] + [pltpu.VMEM((ROW, L), jnp.float32) for _ in range(NBUF)] + [
        pltpu.SemaphoreType.DMA,
        pltpu.SemaphoreType.DMA,
    ],
    compiler_params=_SC_PARAMS,
)
def _sc_edge64(y0, y1, y2, y3, srcR, dstR, acc_out, accS, src_i, dst_i,
               *rest):
    _edge64_body(y0, y1, y2, y3, srcR, dstR, acc_out, accS, src_i, dst_i,
                 *rest)


def _edge8_body(y, srcR, dstR, acc_out, accS, src_i, dst_i, *rest):
    bufs = rest[:NBUF]
    semG, semS = rest[NBUF], rest[NBUF + 1]
    c = lax.axis_index("c")
    s = lax.axis_index("s")
    w = c * NS + s
    myacc = s * NPT

    _zero_acc(accS, bufs[0], myacc)
    plsc.subcore_barrier()
    _sweep(y, srcR, dstR, accS, src_i, dst_i, bufs, semG, semS,
           w * RPT8, ST8)
    plsc.subcore_barrier()
    _dump_acc(accS, bufs[0], acc_out, myacc, c * NP)


@functools.partial(
    pl.kernel,
    out_type=jax.ShapeDtypeStruct((NC * NP, L), jnp.float32),
    mesh=plsc.VectorSubcoreMesh(core_axis_name="c", subcore_axis_name="s"),
    scratch_types=[
        pltpu.VMEM_SHARED((NP, L), jnp.float32),
        pltpu.VMEM((CH, ROW), jnp.int32),
        pltpu.VMEM((CH, ROW), jnp.int32),
    ] + [pltpu.VMEM((ROW, L), jnp.float32) for _ in range(NBUF)] + [
        pltpu.SemaphoreType.DMA,
        pltpu.SemaphoreType.DMA,
    ],
    compiler_params=_SC_PARAMS,
)
def _sc_edge8(y, srcR, dstR, acc_out, accS, src_i, dst_i, *rest):
    _edge8_body(y, srcR, dstR, acc_out, accS, src_i, dst_i, *rest)


NR = NP // ROW // (NC * NS)
SXR = NP // 8 // (NC * NS)
SXN = SXR * 8


def _scalex_body(dinv, d16_out, ds16_out, stage, bufd, bufs):
    c = lax.axis_index("c")
    s = lax.axis_index("s")
    w = c * NS + s

    pltpu.sync_copy(dinv.at[pl.ds(w * SXN, SXN)], stage)

    @pl.loop(0, SXR)
    def _(j):
        for k in range(8):
            idxv = jnp.full((L,), j * 8 + k, jnp.int32)
            vec = plsc.load_gather(stage, [idxv])
            bufd[j, pl.ds(k * L, L)] = vec
            bufs[j, pl.ds(k * L, L)] = vec * vec

    pltpu.sync_copy(bufd, d16_out.at[pl.ds(w * SXR, SXR)])
    pltpu.sync_copy(bufs, ds16_out.at[pl.ds(w * SXR, SXR)])


@functools.partial(
    pl.kernel,
    out_type=[
        jax.ShapeDtypeStruct((NP // 8, ROW), jnp.float32),
        jax.ShapeDtypeStruct((NP // 8, ROW), jnp.float32),
    ],
    mesh=plsc.VectorSubcoreMesh(core_axis_name="c", subcore_axis_name="s"),
    scratch_types=[
        pltpu.VMEM((SXN,), jnp.float32),
        pltpu.VMEM((SXR, ROW), jnp.float32),
        pltpu.VMEM((SXR, ROW), jnp.float32),
    ],
    compiler_params=_SC_PARAMS,
)
def _sc_scalex(dinv, d16_out, ds16_out, stage, bufd, bufs):
    _scalex_body(dinv, d16_out, ds16_out, stage, bufd, bufs)


def _hop(v, mid, out):
    return (v.reshape(mid) + 0.0).reshape(out)


def _t0_body(x_ref, w1_ref, *refs):
    deg_refs = refs[:NC * NS]
    dinv1_ref = refs[NC * NS]
    y_refs = refs[NC * NS + 1:]
    deg = deg_refs[0][...]
    for r in deg_refs[1:]:
        deg = deg + r[...]
    dinv2d = lax.rsqrt(deg + 1.0)
    dinv1_ref[...] = dinv2d.reshape(BLK)
    dinv = _hop(dinv2d, (BR, ROW, 1), (BLK, 1))
    xw = jnp.dot(x_ref[...], w1_ref[...], preferred_element_type=jnp.float32)
    y = xw * dinv
    for f in range(4):
        y_refs[f][...] = _hop(y[:, f * L:(f + 1) * L],
                              (BLK * L // ROW, 8, L), (BLK * L // ROW, ROW))


def _tc_t0(x, W1, degs):
    return pl.pallas_call(
        _t0_body,
        grid=(GRID,),
        in_specs=[
            pl.BlockSpec((BLK, D_IN), lambda i: (i, 0)),
            pl.BlockSpec((D_IN, D_HID), lambda i: (0, 0)),
        ] + [
            pl.BlockSpec((BR, ROW), lambda i, w=w: (w * 50 + i, 0))
            for w in range(NC * NS)
        ],
        out_specs=[
            pl.BlockSpec((BLK,), lambda i: (i,)),
        ] + [pl.BlockSpec((BLK * L // ROW, ROW), lambda i: (i, 0))
             for _ in range(4)],
        out_shape=[
            jax.ShapeDtypeStruct((NP,), jnp.float32),
        ] + [jax.ShapeDtypeStruct((NP * L // ROW, ROW), jnp.float32)
             for _ in range(4)],
    )(x, W1, *degs)


def _mid_body(a0, a1, a2, a3, y0, y1, y2, y3, d_ref, ds_ref, bp_ref, w_ref,
              *out_refs, n_out):
    A = jnp.concatenate([a[...] for a in (a0, a1, a2, a3)], axis=1)
    Y = jnp.concatenate([y[...] for y in (y0, y1, y2, y3)], axis=1)
    D = d_ref[...]
    S = ds_ref[...]
    D4 = jnp.concatenate([D, D, D, D], axis=1)
    S4 = jnp.concatenate([S, S, S, S], axis=1)
    h = jnp.maximum((A + Y) * S4 + bp_ref[...] * D4, 0.0)
    R = jnp.dot(h, w_ref[...], preferred_element_type=jnp.float32)
    for f in range(n_out):
        out_refs[f][...] = R[:, f * ROW:(f + 1) * ROW]


def _tc_mid(acc, ys, d16, ds16, bp, Wp, n_out):
    PB = BLK * L // ROW
    return pl.pallas_call(
        functools.partial(_mid_body, n_out=n_out),
        grid=(GRID,),
        in_specs=[
            pl.BlockSpec((PB, ROW), lambda i, f=f: (f * 50 + i, 0))
            for f in range(4)
        ] + [
            pl.BlockSpec((PB, ROW), lambda i: (i, 0))
            for _ in range(4)
        ] + [
            pl.BlockSpec((PB, ROW), lambda i: (i, 0)),
            pl.BlockSpec((PB, ROW), lambda i: (i, 0)),
            pl.BlockSpec((1, 4 * ROW), lambda i: (0, 0)),
            pl.BlockSpec((4 * ROW, n_out * ROW), lambda i: (0, 0)),
        ],
        out_specs=[pl.BlockSpec((PB, ROW), lambda i: (i, 0))
                   for _ in range(n_out)],
        out_shape=[jax.ShapeDtypeStruct((NP * L // ROW, ROW), jnp.float32)
                   for _ in range(n_out)],
    )(*([acc] * 4), *ys, d16, ds16, bp, Wp)


def _t4_body(a0, a1, y_ref, d_ref, b_ref, out_ref):
    v = (a0[...] + a1[...] + y_ref[...]) * d_ref[...]
    v2 = _hop(v, (BLK * L // ROW, 8, L), (BLK, L))
    out_ref[...] = v2[:, :D_OUT] + b_ref[...]


def _tc_t4(acc3, y3, d16, b3):
    PB = BLK * L // ROW
    return pl.pallas_call(
        _t4_body,
        grid=(GRID,),
        in_specs=[
            pl.BlockSpec((PB, ROW), lambda i, c=c: (c * 50 + i, 0))
            for c in range(NC)
        ] + [
            pl.BlockSpec((PB, ROW), lambda i: (i, 0)),
            pl.BlockSpec((PB, ROW), lambda i: (i, 0)),
            pl.BlockSpec((1, D_OUT), lambda i: (0, 0)),
        ],
        out_specs=pl.BlockSpec((BLK, D_OUT), lambda i: (i, 0)),
        out_shape=jax.ShapeDtypeStruct((N, D_OUT), jnp.float32),
    )(acc3, acc3, y3, d16, b3)


def _pack_w(W, gchunks):
    A = W.reshape(4, L, gchunks, L)
    E = jnp.eye(8, dtype=W.dtype)
    B = A[:, None, :, :, None, :] * E[None, :, None, None, :, None]
    return B.reshape(4 * ROW, gchunks * ROW)


def _pack_b(b):
    return jnp.broadcast_to(b.reshape(4, 1, L), (4, 8, L)).reshape(1, 4 * ROW)


def kernel(x, edge_index, W1, b1, W2, b2, W3, b3):
    src = edge_index[0].astype(jnp.int32)
    dst = edge_index[1].astype(jnp.int32)
    padv = N + jnp.arange(EP - E, dtype=jnp.int32) % NPAD
    srcR = jnp.concatenate([src, padv]).reshape(EROWS, ROW)
    dstR = jnp.concatenate([dst, padv]).reshape(EROWS, ROW)

    W2p = _pack_w(W2, 4)
    W3p = _pack_w(jnp.pad(W3, ((0, 0), (0, L - D_OUT))), 1)
    b1p = _pack_b(b1)
    b2p = _pack_b(b2)

    degs128 = _sc_deg(dstR)
    dinv1, *y1 = _tc_t0(x, W1, [degs128] * (NC * NS))
    d16, ds16 = _sc_scalex(dinv1)

    acc1 = _sc_edge64(*[yc.reshape(NP, L) for yc in y1], srcR, dstR)
    acc1 = acc1.reshape(NP * 4 * L // ROW, ROW)
    y2 = _tc_mid(acc1, y1, d16, ds16, b1p, W2p, 4)

    acc2 = _sc_edge64(*[yc.reshape(NP, L) for yc in y2], srcR, dstR)
    acc2 = acc2.reshape(NP * 4 * L // ROW, ROW)
    (y3,) = _tc_mid(acc2, y2, d16, ds16, b2p, W3p, 1)

    acc3 = _sc_edge8(y3.reshape(NP, L), srcR, dstR)
    acc3 = acc3.reshape(NP * NC * L // ROW, ROW)
    out = _tc_t4(acc3, y3, d16, b3.reshape(1, D_OUT))
    return out


# speedup vs baseline: 28.4086x; 1.0296x over previous
import functools

import jax
import jax.numpy as jnp
from jax import lax
from jax.experimental import pallas as pl
from jax.experimental.pallas import tpu as pltpu
from jax.experimental.pallas import tpu_sc as plsc

N = 100000
E = 1600000
D_IN = 16
D_HID = 64
D_OUT = 8

NP = 102400
ROW = 128
EROWS = 12800
EP = EROWS * ROW
NPAD = NP - N

NC = 2
NS = 16
L = 16

WAVE = 4
NBUF = 2 * WAVE
RPT64 = EROWS // NS
CH = 40
ST64 = RPT64 // CH
NWAVES = CH // WAVE
RPT8 = EROWS // (NC * NS)
ST8 = RPT8 // CH
NPT = NP // NS
DUMP = NPT // ROW
DCH = 50
DST = RPT8 // DCH

BLK = 2048
GRID = (N + BLK - 1) // BLK
BR = BLK // ROW

_SC_PARAMS = pltpu.CompilerParams(
    needs_layout_passes=False, use_tc_tiling_on_sc=False)


def _deg_body(dstR, deg_out, degT, stage):
    c = lax.axis_index("c")
    s = lax.axis_index("s")
    w = c * NS + s
    zeros16 = jnp.zeros((L,), jnp.float32)
    ones16 = jnp.ones((L,), jnp.float32)

    @pl.loop(0, NP // ROW)
    def _(j):
        for k in range(ROW // L):
            degT[j, pl.ds(k * L, L)] = zeros16

    rowbase = w * RPT8
    for st in range(DST):
        pltpu.sync_copy(dstR.at[pl.ds(rowbase + st * DCH, DCH)], stage)

        @pl.loop(0, DCH)
        def _(j):
            row = stage.at[j]
            for k in range(ROW // L):
                idx = row[pl.ds(k * L, L)]
                plsc.addupdate_scatter(
                    degT,
                    [lax.shift_right_logical(idx, 7),
                     lax.bitwise_and(idx, 127)],
                    ones16)

    pltpu.sync_copy(degT, deg_out.at[pl.ds(w * (NP // ROW), NP // ROW)])


@functools.partial(
    pl.kernel,
    out_type=jax.ShapeDtypeStruct((NC * NS * (NP // ROW), ROW), jnp.float32),
    mesh=plsc.VectorSubcoreMesh(core_axis_name="c", subcore_axis_name="s"),
    scratch_types=[
        pltpu.VMEM((NP // ROW, ROW), jnp.float32),
        pltpu.VMEM((DCH, ROW), jnp.int32),
    ],
    compiler_params=_SC_PARAMS,
)
def _sc_deg(dstR, deg_out, degT, stage):
    _deg_body(dstR, deg_out, degT, stage)


def _sweep(ytab, srcR, dstR, accS, src_i, dst_i, bufs, semG, semS,
           rowbase, nstages):
    @pl.loop(0, nstages)
    def _(st):
        r0 = rowbase + st * CH
        pltpu.sync_copy(srcR.at[pl.ds(r0, CH)], src_i)
        pltpu.sync_copy(dstR.at[pl.ds(r0, CH)], dst_i)

        gd = {}
        sd = {}
        gd[0] = [
            pltpu.async_copy(ytab.at[src_i.at[b]], bufs[b], semG)
            for b in range(WAVE)
        ]
        for w in range(NWAVES):
            grp = [bufs[(w % 2) * WAVE + b] for b in range(WAVE)]
            nxt = [bufs[((w + 1) % 2) * WAVE + b] for b in range(WAVE)]
            for d in gd[w]:
                d.wait()
            if w >= 1:
                for d in sd[w - 1]:
                    d.wait()
            if w + 1 < NWAVES:
                gd[w + 1] = [
                    pltpu.async_copy(
                        ytab.at[src_i.at[(w + 1) * WAVE + b]], nxt[b], semG)
                    for b in range(WAVE)
                ]
            sd[w] = [
                pltpu.async_copy(
                    grp[b], accS.at[dst_i.at[w * WAVE + b]], semS, add=True)
                for b in range(WAVE)
            ]
        for d in sd[NWAVES - 1]:
            d.wait()


def _zero_acc(accS, zbuf, myacc, sem):
    zeros16 = jnp.zeros((L,), jnp.float32)

    @pl.loop(0, ROW)
    def _(j):
        zbuf[j] = zeros16

    @pl.loop(0, DUMP)
    def _(k):
        pltpu.async_copy(zbuf, accS.at[pl.ds(myacc + k * ROW, ROW)], sem)

    @pl.loop(0, DUMP)
    def _(k):
        pltpu.make_async_copy(
            zbuf, accS.at[pl.ds(myacc + k * ROW, ROW)], sem).wait()


def _dump_acc(accS, bufs, acc_out, myacc, out_base, semI, semO):
    @pl.loop(0, DUMP // 5)
    def _(g):
        base = myacc + g * 5 * ROW
        obase = out_base + base
        ins = [pltpu.async_copy(accS.at[pl.ds(base + q * ROW, ROW)],
                                bufs[q], semI) for q in range(5)]
        outs = []
        for q in range(5):
            ins[q].wait()
            outs.append(pltpu.async_copy(
                bufs[q], acc_out.at[pl.ds(obase + q * ROW, ROW)], semO))
        for d in outs:
            d.wait()


def _edge64_body(y0, y1, y2, y3, srcR, dstR, acc_out, accS, src_i, dst_i,
                 *rest):
    bufs = rest[:NBUF]
    semG, semS = rest[NBUF], rest[NBUF + 1]
    c = lax.axis_index("c")
    s = lax.axis_index("s")
    rowbase = s * RPT64
    myacc = s * NPT

    def one_pass(ytab, out_base):
        _zero_acc(accS, bufs[0], myacc, semG)
        plsc.subcore_barrier()
        _sweep(ytab, srcR, dstR, accS, src_i, dst_i, bufs, semG, semS,
               rowbase, ST64)
        plsc.subcore_barrier()
        _dump_acc(accS, bufs[:5], acc_out, myacc, out_base, semG, semS)
        plsc.subcore_barrier()

    @pl.when(c == 0)
    def _():
        one_pass(y0, 0)
        one_pass(y1, NP)

    @pl.when(c == 1)
    def _():
        one_pass(y2, 2 * NP)
        one_pass(y3, 3 * NP)


@functools.partial(
    pl.kernel,
    out_type=jax.ShapeDtypeStruct((4 * NP, L), jnp.float32),
    mesh=plsc.VectorSubcoreMesh(core_axis_name="c", subcore_axis_name="s"),
    scratch_types=[
        pltpu.VMEM_SHARED((NP, L), jnp.float32),
        pltpu.VMEM((CH, ROW), jnp.int32),
        pltpu.VMEM((CH, ROW), jnp.int32),
    ] + [pltpu.VMEM((ROW, L), jnp.float32) for _ in range(NBUF)] + [
        pltpu.SemaphoreType.DMA,
        pltpu.SemaphoreType.DMA,
    ],
    compiler_params=_SC_PARAMS,
)
def _sc_edge64(y0, y1, y2, y3, srcR, dstR, acc_out, accS, src_i, dst_i,
               *rest):
    _edge64_body(y0, y1, y2, y3, srcR, dstR, acc_out, accS, src_i, dst_i,
                 *rest)


def _edge8_body(y, srcR, dstR, acc_out, accS, src_i, dst_i, *rest):
    bufs = rest[:NBUF]
    semG, semS = rest[NBUF], rest[NBUF + 1]
    c = lax.axis_index("c")
    s = lax.axis_index("s")
    w = c * NS + s
    myacc = s * NPT

    _zero_acc(accS, bufs[0], myacc, semG)
    plsc.subcore_barrier()
    _sweep(y, srcR, dstR, accS, src_i, dst_i, bufs, semG, semS,
           w * RPT8, ST8)
    plsc.subcore_barrier()
    _dump_acc(accS, bufs[:5], acc_out, myacc, c * NP, semG, semS)


@functools.partial(
    pl.kernel,
    out_type=jax.ShapeDtypeStruct((NC * NP, L), jnp.float32),
    mesh=plsc.VectorSubcoreMesh(core_axis_name="c", subcore_axis_name="s"),
    scratch_types=[
        pltpu.VMEM_SHARED((NP, L), jnp.float32),
        pltpu.VMEM((CH, ROW), jnp.int32),
        pltpu.VMEM((CH, ROW), jnp.int32),
    ] + [pltpu.VMEM((ROW, L), jnp.float32) for _ in range(NBUF)] + [
        pltpu.SemaphoreType.DMA,
        pltpu.SemaphoreType.DMA,
    ],
    compiler_params=_SC_PARAMS,
)
def _sc_edge8(y, srcR, dstR, acc_out, accS, src_i, dst_i, *rest):
    _edge8_body(y, srcR, dstR, acc_out, accS, src_i, dst_i, *rest)


NR = NP // ROW // (NC * NS)
SXR = NP // 8 // (NC * NS)
SXN = SXR * 8


def _scalex_body(dinv, d16_out, ds16_out, stage, bufd, bufs):
    c = lax.axis_index("c")
    s = lax.axis_index("s")
    w = c * NS + s

    pltpu.sync_copy(dinv.at[pl.ds(w * SXN, SXN)], stage)

    @pl.loop(0, SXR)
    def _(j):
        for k in range(8):
            idxv = jnp.full((L,), j * 8 + k, jnp.int32)
            vec = plsc.load_gather(stage, [idxv])
            bufd[j, pl.ds(k * L, L)] = vec
            bufs[j, pl.ds(k * L, L)] = vec * vec

    pltpu.sync_copy(bufd, d16_out.at[pl.ds(w * SXR, SXR)])
    pltpu.sync_copy(bufs, ds16_out.at[pl.ds(w * SXR, SXR)])


@functools.partial(
    pl.kernel,
    out_type=[
        jax.ShapeDtypeStruct((NP // 8, ROW), jnp.float32),
        jax.ShapeDtypeStruct((NP // 8, ROW), jnp.float32),
    ],
    mesh=plsc.VectorSubcoreMesh(core_axis_name="c", subcore_axis_name="s"),
    scratch_types=[
        pltpu.VMEM((SXN,), jnp.float32),
        pltpu.VMEM((SXR, ROW), jnp.float32),
        pltpu.VMEM((SXR, ROW), jnp.float32),
    ],
    compiler_params=_SC_PARAMS,
)
def _sc_scalex(dinv, d16_out, ds16_out, stage, bufd, bufs):
    _scalex_body(dinv, d16_out, ds16_out, stage, bufd, bufs)


def _hop(v, mid, out):
    return (v.reshape(mid) + 0.0).reshape(out)


def _t0_body(x_ref, w1_ref, *refs):
    deg_refs = refs[:NC * NS]
    dinv1_ref = refs[NC * NS]
    y_refs = refs[NC * NS + 1:]
    deg = deg_refs[0][...]
    for r in deg_refs[1:]:
        deg = deg + r[...]
    dinv2d = lax.rsqrt(deg + 1.0)
    dinv1_ref[...] = dinv2d.reshape(BLK)
    dinv = _hop(dinv2d, (BR, ROW, 1), (BLK, 1))
    xw = jnp.dot(x_ref[...], w1_ref[...], preferred_element_type=jnp.float32)
    y = xw * dinv
    for f in range(4):
        y_refs[f][...] = _hop(y[:, f * L:(f + 1) * L],
                              (BLK * L // ROW, 8, L), (BLK * L // ROW, ROW))


def _tc_t0(x, W1, degs):
    return pl.pallas_call(
        _t0_body,
        grid=(GRID,),
        in_specs=[
            pl.BlockSpec((BLK, D_IN), lambda i: (i, 0)),
            pl.BlockSpec((D_IN, D_HID), lambda i: (0, 0)),
        ] + [
            pl.BlockSpec((BR, ROW), lambda i, w=w: (w * 50 + i, 0))
            for w in range(NC * NS)
        ],
        out_specs=[
            pl.BlockSpec((BLK,), lambda i: (i,)),
        ] + [pl.BlockSpec((BLK * L // ROW, ROW), lambda i: (i, 0))
             for _ in range(4)],
        out_shape=[
            jax.ShapeDtypeStruct((NP,), jnp.float32),
        ] + [jax.ShapeDtypeStruct((NP * L // ROW, ROW), jnp.float32)
             for _ in range(4)],
    )(x, W1, *degs)


def _mid_body(a0, a1, a2, a3, y0, y1, y2, y3, d_ref, ds_ref, bp_ref, w_ref,
              *out_refs, n_out):
    A = jnp.concatenate([a[...] for a in (a0, a1, a2, a3)], axis=1)
    Y = jnp.concatenate([y[...] for y in (y0, y1, y2, y3)], axis=1)
    D = d_ref[...]
    S = ds_ref[...]
    D4 = jnp.concatenate([D, D, D, D], axis=1)
    S4 = jnp.concatenate([S, S, S, S], axis=1)
    h = jnp.maximum((A + Y) * S4 + bp_ref[...] * D4, 0.0)
    R = jnp.dot(h, w_ref[...], preferred_element_type=jnp.float32)
    for f in range(n_out):
        out_refs[f][...] = R[:, f * ROW:(f + 1) * ROW]


def _tc_mid(acc, ys, d16, ds16, bp, Wp, n_out):
    PB = BLK * L // ROW
    return pl.pallas_call(
        functools.partial(_mid_body, n_out=n_out),
        grid=(GRID,),
        in_specs=[
            pl.BlockSpec((PB, ROW), lambda i, f=f: (f * 50 + i, 0))
            for f in range(4)
        ] + [
            pl.BlockSpec((PB, ROW), lambda i: (i, 0))
            for _ in range(4)
        ] + [
            pl.BlockSpec((PB, ROW), lambda i: (i, 0)),
            pl.BlockSpec((PB, ROW), lambda i: (i, 0)),
            pl.BlockSpec((1, 4 * ROW), lambda i: (0, 0)),
            pl.BlockSpec((4 * ROW, n_out * ROW), lambda i: (0, 0)),
        ],
        out_specs=[pl.BlockSpec((PB, ROW), lambda i: (i, 0))
                   for _ in range(n_out)],
        out_shape=[jax.ShapeDtypeStruct((NP * L // ROW, ROW), jnp.float32)
                   for _ in range(n_out)],
    )(*([acc] * 4), *ys, d16, ds16, bp, Wp)


def _t4_body(a0, a1, y_ref, d_ref, b_ref, out_ref):
    v = (a0[...] + a1[...] + y_ref[...]) * d_ref[...]
    v2 = _hop(v, (BLK * L // ROW, 8, L), (BLK, L))
    out_ref[...] = v2[:, :D_OUT] + b_ref[...]


def _tc_t4(acc3, y3, d16, b3):
    PB = BLK * L // ROW
    return pl.pallas_call(
        _t4_body,
        grid=(GRID,),
        in_specs=[
            pl.BlockSpec((PB, ROW), lambda i, c=c: (c * 50 + i, 0))
            for c in range(NC)
        ] + [
            pl.BlockSpec((PB, ROW), lambda i: (i, 0)),
            pl.BlockSpec((PB, ROW), lambda i: (i, 0)),
            pl.BlockSpec((1, D_OUT), lambda i: (0, 0)),
        ],
        out_specs=pl.BlockSpec((BLK, D_OUT), lambda i: (i, 0)),
        out_shape=jax.ShapeDtypeStruct((N, D_OUT), jnp.float32),
    )(acc3, acc3, y3, d16, b3)


def _pack_w(W, gchunks):
    A = W.reshape(4, L, gchunks, L)
    E = jnp.eye(8, dtype=W.dtype)
    B = A[:, None, :, :, None, :] * E[None, :, None, None, :, None]
    return B.reshape(4 * ROW, gchunks * ROW)


def _pack_b(b):
    return jnp.broadcast_to(b.reshape(4, 1, L), (4, 8, L)).reshape(1, 4 * ROW)


def kernel(x, edge_index, W1, b1, W2, b2, W3, b3):
    src = edge_index[0].astype(jnp.int32)
    dst = edge_index[1].astype(jnp.int32)
    padv = N + jnp.arange(EP - E, dtype=jnp.int32) % NPAD
    srcR = jnp.concatenate([src, padv]).reshape(EROWS, ROW)
    dstR = jnp.concatenate([dst, padv]).reshape(EROWS, ROW)

    W2p = _pack_w(W2, 4)
    W3p = _pack_w(jnp.pad(W3, ((0, 0), (0, L - D_OUT))), 1)
    b1p = _pack_b(b1)
    b2p = _pack_b(b2)

    degs128 = _sc_deg(dstR)
    dinv1, *y1 = _tc_t0(x, W1, [degs128] * (NC * NS))
    d16, ds16 = _sc_scalex(dinv1)

    acc1 = _sc_edge64(*[yc.reshape(NP, L) for yc in y1], srcR, dstR)
    acc1 = acc1.reshape(NP * 4 * L // ROW, ROW)
    y2 = _tc_mid(acc1, y1, d16, ds16, b1p, W2p, 4)

    acc2 = _sc_edge64(*[yc.reshape(NP, L) for yc in y2], srcR, dstR)
    acc2 = acc2.reshape(NP * 4 * L // ROW, ROW)
    (y3,) = _tc_mid(acc2, y2, d16, ds16, b2p, W3p, 1)

    acc3 = _sc_edge8(y3.reshape(NP, L), srcR, dstR)
    acc3 = acc3.reshape(NP * NC * L // ROW, ROW)
    out = _tc_t4(acc3, y3, d16, b3.reshape(1, D_OUT))
    return out
